# baseline (speedup 1.0000x reference)
import functools

import jax
import jax.numpy as jnp
from jax import lax
from jax.experimental import pallas as pl
from jax.experimental.pallas import tpu as pltpu
from jax.experimental.pallas import tpu_sc as plsc

_TAU = 2048


def _build_sc_kernel(B, E, H, L, NC):
    tail = _TAU * H
    out_e = E + tail
    NPIPE = 4
    C = E // NPIPE
    period = H * L
    nper = tail // period
    assert tail % period == 0 and E % NPIPE == 0

    shift = 16
    mult = -(-(1 << shift) // H)
    for j in range(period):
        assert (j * mult) >> shift == j // H

    mesh = plsc.VectorSubcoreMesh(core_axis_name="c", subcore_axis_name="s")

    @functools.partial(
        pl.kernel,
        mesh=mesh,
        out_type=(
            jax.ShapeDtypeStruct((B, 2, out_e), jnp.int32),
            jax.ShapeDtypeStruct((B, 1, out_e), jnp.int32),
        ),
        scratch_types=[
            pltpu.VMEM((E,), jnp.int32),
            pltpu.VMEM((3 * tail,), jnp.int32),
            pltpu.VMEM((B + H, L), jnp.int32),
            pltpu.SemaphoreType.DMA,
            pltpu.SemaphoreType.DMA,
            pltpu.SemaphoreType.DMA,
            pltpu.SemaphoreType.DMA,
            pltpu.SemaphoreType.DMA,
        ],
    )
    def sc_k(e_hbm, w_hbm, params_hbm, eout_hbm, wout_hbm, buf, tl, par_v,
             s0, s1, s2, s3, sem_o):
        c = lax.axis_index("c")
        s = lax.axis_index("s")
        w = s * NC + c
        sems = [s0, s1, s2, s3]

        def copy_row(src, dst):
            for k in range(NPIPE):
                pltpu.async_copy(src(k), buf.at[pl.ds(k * C, C)], sems[k])
            for k in range(NPIPE):
                pltpu.make_async_copy(
                    src(k), buf.at[pl.ds(k * C, C)], sems[k]
                ).wait()
                pltpu.async_copy(buf.at[pl.ds(k * C, C)], dst(k), sem_o)
            for k in range(NPIPE):
                pltpu.make_async_copy(
                    buf.at[pl.ds(k * C, C)], dst(k), sem_o
                ).wait()

        @pl.when(w < 2 * B)
        def _edge_copy():
            b = lax.div(w, 2)
            i = lax.rem(w, 2)
            copy_row(
                lambda k: e_hbm.at[b, i, pl.ds(k * C, C)],
                lambda k: eout_hbm.at[b, i, pl.ds(k * C, C)],
            )

        @pl.when(jnp.logical_and(w >= 2 * B, w < 3 * B))
        def _weight_copy():
            b = w - 2 * B
            copy_row(
                lambda k: w_hbm.at[b, 0, pl.ds(k * C, C)],
                lambda k: wout_hbm.at[b, 0, pl.ds(k * C, C)],
            )

        @pl.when(w >= 3 * B)
        def _tails():
            b = w - 3 * B
            pltpu.sync_copy(params_hbm, par_v)
            base_v = par_v[b]
            lanes = lax.broadcasted_iota(jnp.int32, (L,), 0)
            zero = jnp.zeros((L,), jnp.int32)
            ends = []
            starts = []
            for h in range(H):
                j = h * L + lanes
                t = (j * mult) >> shift
                r = j - t * H
                hop = par_v[B + H - 1]
                for hh in range(H - 2, -1, -1):
                    hop = jnp.where(r == hh, par_v[B + hh], hop)
                ends.append(base_v + t)
                starts.append(base_v + t - hop)

            def body(ci, carry):
                off = ci * period
                es, ss = carry[:H], carry[H:]
                for h in range(H):
                    o = off + h * L
                    tl[pl.ds(o, L)] = es[h]
                    tl[pl.ds(tail + o, L)] = ss[h]
                    tl[pl.ds(2 * tail + o, L)] = zero
                return tuple(v + L for v in carry)

            lax.fori_loop(0, nper, body, tuple(ends) + tuple(starts))

            pltpu.async_copy(
                tl.at[pl.ds(0, tail)], eout_hbm.at[b, 0, pl.ds(E, tail)], sem_o
            )
            pltpu.async_copy(
                tl.at[pl.ds(tail, tail)], eout_hbm.at[b, 1, pl.ds(E, tail)], sem_o
            )
            pltpu.async_copy(
                tl.at[pl.ds(2 * tail, tail)], wout_hbm.at[b, 0, pl.ds(E, tail)], sem_o
            )
            for kk in range(3):
                pltpu.make_async_copy(
                    tl.at[pl.ds(kk * tail, tail)],
                    eout_hbm.at[0, 0, pl.ds(E, tail)],
                    sem_o,
                ).wait()

    return sc_k


def kernel(nodes, edges, weights, T, taus, hops):
    del nodes
    B, _, E = edges.shape
    H = hops.shape[0]
    edtype = edges.dtype

    info = plsc.get_sparse_core_info()
    NC, L = info.num_cores, info.num_lanes

    base = T.astype(jnp.int32) + taus.astype(jnp.int32) - _TAU
    scal = jnp.concatenate([base, hops.astype(jnp.int32)])
    params = jnp.broadcast_to(scal[:, None], (B + H, L))

    sc_k = _build_sc_kernel(B, E, H, L, NC)
    edges_out, weights_bits = sc_k(
        edges.astype(jnp.int32),
        lax.bitcast_convert_type(weights, jnp.int32),
        params,
    )
    weights_out = lax.bitcast_convert_type(weights_bits, weights.dtype)
    return edges_out.astype(edtype), weights_out


# speedup vs baseline: 1.0202x; 1.0202x over previous
import functools

import jax
import jax.numpy as jnp
from jax import lax
from jax.experimental import pallas as pl
from jax.experimental.pallas import tpu as pltpu
from jax.experimental.pallas import tpu_sc as plsc

_TAU = 2048


def _build_sc_kernel(B, E, H, L, NC, NS):
    NW = NC * NS
    tail = _TAU * H
    out_e = E + tail
    R = 3 * B
    NCH = 4 * R
    CPW = NCH // NW
    C = E // 4
    period = H * L
    nper = tail // period
    assert tail % period == 0 and NCH % NW == 0 and E % 4 == 0

    shift = 16
    mult = -(-(1 << shift) // H)
    for j in range(period):
        assert (j * mult) >> shift == j // H

    mesh = plsc.VectorSubcoreMesh(core_axis_name="c", subcore_axis_name="s")

    @functools.partial(
        pl.kernel,
        mesh=mesh,
        out_type=(
            jax.ShapeDtypeStruct((B, 2, out_e), jnp.int32),
            jax.ShapeDtypeStruct((B, 1, out_e), jnp.int32),
        ),
        scratch_types=[
            pltpu.VMEM((CPW * C,), jnp.int32),
            pltpu.VMEM((tail,), jnp.int32),
            pltpu.VMEM((B + H, L), jnp.int32),
            pltpu.SemaphoreType.DMA,
            pltpu.SemaphoreType.DMA,
            pltpu.SemaphoreType.DMA,
            pltpu.SemaphoreType.DMA,
        ],
    )
    def sc_k(e_hbm, w_hbm, params_hbm, eout_hbm, wout_hbm, buf, tl, par_v,
             s0, s1, s2, sem_o):
        c = lax.axis_index("c")
        s = lax.axis_index("s")
        w = s * NC + c
        b = lax.div(w, 3)
        kind = lax.rem(w, 3)
        is_edge_tail = jnp.logical_and(w < R, kind < 2)
        is_wt_tail = jnp.logical_and(w < R, kind == 2)
        sems = [s0, s1, s2]

        def chunk_coords(qi):
            q = w + NW * qi
            row = lax.div(q, 4)
            part = lax.rem(q, 4)
            return lax.div(row, 3), lax.rem(row, 3), part

        for qi in range(CPW):
            qb, qk, part = chunk_coords(qi)

            @pl.when(qk < 2)
            def _(qi=qi, qb=qb, qk=qk, part=part):
                pltpu.async_copy(
                    e_hbm.at[qb, qk, pl.ds(part * C, C)],
                    buf.at[pl.ds(qi * C, C)], sems[qi]
                )

            @pl.when(qk == 2)
            def _(qi=qi, qb=qb, part=part):
                pltpu.async_copy(
                    w_hbm.at[qb, 0, pl.ds(part * C, C)],
                    buf.at[pl.ds(qi * C, C)], sems[qi]
                )

        @pl.when(is_edge_tail)
        def _edge_tail():
            pltpu.sync_copy(params_hbm, par_v)
            base_v = par_v[b]
            kind_v = jnp.full((L,), kind, jnp.int32)
            lanes = lax.broadcasted_iota(jnp.int32, (L,), 0)
            seeds = []
            for h in range(H):
                j = h * L + lanes
                t = (j * mult) >> shift
                r = j - t * H
                hop = par_v[B + H - 1]
                for hh in range(H - 2, -1, -1):
                    hop = jnp.where(r == hh, par_v[B + hh], hop)
                seeds.append(base_v + t - kind_v * hop)

            def body(ci, carry):
                off = ci * period
                for h in range(H):
                    tl[pl.ds(off + h * L, L)] = carry[h]
                return tuple(v + L for v in carry)

            lax.fori_loop(0, nper, body, tuple(seeds))

        @pl.when(is_wt_tail)
        def _weight_tail():
            zero = jnp.zeros((L,), jnp.int32)

            def zbody(ci, carry):
                off = ci * period
                for h in range(H):
                    tl[pl.ds(off + h * L, L)] = zero
                return carry

            lax.fori_loop(0, nper, zbody, 0)

        for qi in range(CPW):
            qb, qk, part = chunk_coords(qi)
            pltpu.make_async_copy(
                e_hbm.at[0, 0, pl.ds(0, C)],
                buf.at[pl.ds(qi * C, C)], sems[qi]
            ).wait()

            @pl.when(qk < 2)
            def _(qi=qi, qb=qb, qk=qk, part=part):
                pltpu.async_copy(
                    buf.at[pl.ds(qi * C, C)],
                    eout_hbm.at[qb, qk, pl.ds(part * C, C)], sem_o
                )

            @pl.when(qk == 2)
            def _(qi=qi, qb=qb, part=part):
                pltpu.async_copy(
                    buf.at[pl.ds(qi * C, C)],
                    wout_hbm.at[qb, 0, pl.ds(part * C, C)], sem_o
                )

        @pl.when(is_edge_tail)
        def _edge_tail_out():
            pltpu.async_copy(tl, eout_hbm.at[b, kind, pl.ds(E, tail)], sem_o)

        @pl.when(is_wt_tail)
        def _weight_tail_out():
            pltpu.async_copy(tl, wout_hbm.at[b, 0, pl.ds(E, tail)], sem_o)

        for qi in range(CPW):
            pltpu.make_async_copy(
                buf.at[pl.ds(qi * C, C)], eout_hbm.at[0, 0, pl.ds(0, C)], sem_o
            ).wait()

        @pl.when(w < R)
        def _tail_drain():
            pltpu.make_async_copy(
                tl, eout_hbm.at[0, 0, pl.ds(E, tail)], sem_o
            ).wait()

    return sc_k


def kernel(nodes, edges, weights, T, taus, hops):
    del nodes
    B, _, E = edges.shape
    H = hops.shape[0]
    edtype = edges.dtype

    info = plsc.get_sparse_core_info()
    NC, NS, L = info.num_cores, info.num_subcores, info.num_lanes

    base = T.astype(jnp.int32) + taus.astype(jnp.int32) - _TAU
    scal = jnp.concatenate([base, hops.astype(jnp.int32)])
    params = jnp.broadcast_to(scal[:, None], (B + H, L))

    sc_k = _build_sc_kernel(B, E, H, L, NC, NS)
    edges_out, weights_bits = sc_k(
        edges.astype(jnp.int32),
        lax.bitcast_convert_type(weights, jnp.int32),
        params,
    )
    weights_out = lax.bitcast_convert_type(weights_bits, weights.dtype)
    return edges_out.astype(edtype), weights_out


# speedup vs baseline: 1.1842x; 1.1607x over previous
import functools

import jax
import jax.numpy as jnp
from jax import lax
from jax.experimental import pallas as pl
from jax.experimental.pallas import tpu as pltpu
from jax.experimental.pallas import tpu_sc as plsc

_TAU = 2048


def _build_sc_edges_kernel(B, E, H, L, NC):
    tail = _TAU * H
    out_e = E + tail
    HALF = E // 2
    C = HALF // 2
    period = H * L
    nper = tail // period
    assert tail % period == 0 and E % 4 == 0

    shift = 16
    mult = -(-(1 << shift) // H)
    for j in range(period):
        assert (j * mult) >> shift == j // H

    mesh = plsc.VectorSubcoreMesh(core_axis_name="c", subcore_axis_name="s")

    @functools.partial(
        pl.kernel,
        mesh=mesh,
        out_type=jax.ShapeDtypeStruct((B, 2, out_e), jnp.int32),
        scratch_types=[
            pltpu.VMEM((HALF,), jnp.int32),
            pltpu.VMEM((tail,), jnp.int32),
            pltpu.VMEM((B + H, L), jnp.int32),
            pltpu.SemaphoreType.DMA,
            pltpu.SemaphoreType.DMA,
            pltpu.SemaphoreType.DMA,
        ],
    )
    def sc_k(e_hbm, params_hbm, eout_hbm, buf, tl, par_v, s0, s1, sem_o):
        c = lax.axis_index("c")
        s = lax.axis_index("s")
        w = s * NC + c
        row = lax.div(w, 2)
        b = lax.div(row, 2)
        i = lax.rem(row, 2)
        half = lax.rem(w, 2)
        off = half * HALF
        do_tail = lax.rem(w, 2) == lax.rem(row, 2)
        sems = [s0, s1]

        for k in range(2):
            pltpu.async_copy(
                e_hbm.at[b, i, pl.ds(off + k * C, C)],
                buf.at[pl.ds(k * C, C)], sems[k]
            )

        @pl.when(do_tail)
        def _gen_tail():
            pltpu.sync_copy(params_hbm, par_v)
            base_v = par_v[b]
            i_v = jnp.full((L,), i, jnp.int32)
            lanes = lax.broadcasted_iota(jnp.int32, (L,), 0)
            seeds = []
            for h in range(H):
                j = h * L + lanes
                t = (j * mult) >> shift
                r = j - t * H
                hop = par_v[B + H - 1]
                for hh in range(H - 2, -1, -1):
                    hop = jnp.where(r == hh, par_v[B + hh], hop)
                seeds.append(base_v + t - i_v * hop)

            def body(ci, carry):
                o = ci * period
                for h in range(H):
                    tl[pl.ds(o + h * L, L)] = carry[h]
                return tuple(v + L for v in carry)

            lax.fori_loop(0, nper, body, tuple(seeds))

        for k in range(2):
            pltpu.make_async_copy(
                e_hbm.at[0, 0, pl.ds(0, C)], buf.at[pl.ds(k * C, C)], sems[k]
            ).wait()
            pltpu.async_copy(
                buf.at[pl.ds(k * C, C)],
                eout_hbm.at[b, i, pl.ds(off + k * C, C)], sem_o
            )

        @pl.when(do_tail)
        def _tail_out():
            pltpu.async_copy(tl, eout_hbm.at[b, i, pl.ds(E, tail)], sem_o)

        for k in range(2):
            pltpu.make_async_copy(
                buf.at[pl.ds(k * C, C)], eout_hbm.at[0, 0, pl.ds(0, C)], sem_o
            ).wait()

        @pl.when(do_tail)
        def _tail_drain():
            pltpu.make_async_copy(
                tl, eout_hbm.at[0, 0, pl.ds(E, tail)], sem_o
            ).wait()

    return sc_k


def _build_tc_weights_kernel(B, E, H, wdtype):
    tail = _TAU * H
    out_e = E + tail

    def body(w_ref, o_ref):
        o_ref[:, :, pl.ds(0, E)] = w_ref[...]
        o_ref[:, :, pl.ds(E, tail)] = jnp.zeros((1, 1, tail), wdtype)

    return pl.pallas_call(
        body,
        grid=(B,),
        in_specs=[pl.BlockSpec((1, 1, E), lambda b: (b, 0, 0))],
        out_specs=pl.BlockSpec((1, 1, out_e), lambda b: (b, 0, 0)),
        out_shape=jax.ShapeDtypeStruct((B, 1, out_e), wdtype),
    )


def kernel(nodes, edges, weights, T, taus, hops):
    del nodes
    B, _, E = edges.shape
    H = hops.shape[0]
    edtype = edges.dtype

    info = plsc.get_sparse_core_info()
    NC, L = info.num_cores, info.num_lanes

    base = T.astype(jnp.int32) + taus.astype(jnp.int32) - _TAU
    scal = jnp.concatenate([base, hops.astype(jnp.int32)])
    params = jnp.broadcast_to(scal[:, None], (B + H, L))

    sc_k = _build_sc_edges_kernel(B, E, H, L, NC)
    edges_out = sc_k(edges.astype(jnp.int32), params)
    weights_out = _build_tc_weights_kernel(B, E, H, weights.dtype)(weights)
    return edges_out.astype(edtype), weights_out


# speedup vs baseline: 1.1953x; 1.0094x over previous
import functools

import jax
import jax.numpy as jnp
from jax import lax
from jax.experimental import pallas as pl
from jax.experimental.pallas import tpu as pltpu
from jax.experimental.pallas import tpu_sc as plsc

_TAU = 2048


def _build_sc_edges_kernel(B, E, H, L, NC):
    tail = _TAU * H
    out_e = E + tail
    HALF = E // 2
    NPIPE = 4
    C = HALF // NPIPE
    period = H * L
    nper = tail // period
    assert tail % period == 0 and E % 4 == 0

    shift = 16
    mult = -(-(1 << shift) // H)
    for j in range(period):
        assert (j * mult) >> shift == j // H

    mesh = plsc.VectorSubcoreMesh(core_axis_name="c", subcore_axis_name="s")

    @functools.partial(
        pl.kernel,
        mesh=mesh,
        out_type=jax.ShapeDtypeStruct((B, 2, out_e), jnp.int32),
        scratch_types=[
            pltpu.VMEM((HALF,), jnp.int32),
            pltpu.VMEM((tail,), jnp.int32),
            pltpu.VMEM((B + H, L), jnp.int32),
            pltpu.SemaphoreType.DMA,
            pltpu.SemaphoreType.DMA,
            pltpu.SemaphoreType.DMA,
            pltpu.SemaphoreType.DMA,
            pltpu.SemaphoreType.DMA,
            pltpu.SemaphoreType.DMA,
        ],
    )
    def sc_k(e_hbm, params_hbm, eout_hbm, buf, tl, par_v,
             s0, s1, s2, s3, sem_p, sem_o):
        c = lax.axis_index("c")
        s = lax.axis_index("s")
        w = s * NC + c
        row = lax.div(w, 2)
        b = lax.div(row, 2)
        i = lax.rem(row, 2)
        half = lax.rem(w, 2)
        off = half * HALF
        do_tail = lax.rem(w, 2) == lax.rem(row, 2)
        sems = [s0, s1, s2, s3]

        @pl.when(do_tail)
        def _params():
            pltpu.async_copy(params_hbm, par_v, sem_p)

        for k in range(NPIPE):
            pltpu.async_copy(
                e_hbm.at[b, i, pl.ds(off + k * C, C)],
                buf.at[pl.ds(k * C, C)], sems[k]
            )

        for k in range(NPIPE):
            pltpu.make_async_copy(
                e_hbm.at[0, 0, pl.ds(0, C)], buf.at[pl.ds(k * C, C)], sems[k]
            ).wait()
            pltpu.async_copy(
                buf.at[pl.ds(k * C, C)],
                eout_hbm.at[b, i, pl.ds(off + k * C, C)], sem_o
            )

        @pl.when(do_tail)
        def _gen_tail():
            pltpu.make_async_copy(params_hbm, par_v, sem_p).wait()
            base_v = par_v[b]
            i_v = jnp.full((L,), i, jnp.int32)
            lanes = lax.broadcasted_iota(jnp.int32, (L,), 0)
            seeds = []
            for h in range(H):
                j = h * L + lanes
                t = (j * mult) >> shift
                r = j - t * H
                hop = par_v[B + H - 1]
                for hh in range(H - 2, -1, -1):
                    hop = jnp.where(r == hh, par_v[B + hh], hop)
                seeds.append(base_v + t - i_v * hop)

            def body(ci, carry):
                o = ci * period
                for h in range(H):
                    tl[pl.ds(o + h * L, L)] = carry[h]
                return tuple(v + L for v in carry)

            lax.fori_loop(0, nper, body, tuple(seeds))
            pltpu.async_copy(tl, eout_hbm.at[b, i, pl.ds(E, tail)], sem_o)

        for k in range(NPIPE):
            pltpu.make_async_copy(
                buf.at[pl.ds(k * C, C)], eout_hbm.at[0, 0, pl.ds(0, C)], sem_o
            ).wait()

        @pl.when(do_tail)
        def _tail_drain():
            pltpu.make_async_copy(
                tl, eout_hbm.at[0, 0, pl.ds(E, tail)], sem_o
            ).wait()

    return sc_k


def _build_tc_weights_kernel(B, E, H, wdtype):
    tail = _TAU * H
    out_e = E + tail

    def body(w_ref, o_ref):
        o_ref[:, :, pl.ds(0, E)] = w_ref[...]
        o_ref[:, :, pl.ds(E, tail)] = jnp.zeros((1, 1, tail), wdtype)

    return pl.pallas_call(
        body,
        grid=(B,),
        in_specs=[pl.BlockSpec((1, 1, E), lambda b: (b, 0, 0))],
        out_specs=pl.BlockSpec((1, 1, out_e), lambda b: (b, 0, 0)),
        out_shape=jax.ShapeDtypeStruct((B, 1, out_e), wdtype),
    )


def kernel(nodes, edges, weights, T, taus, hops):
    del nodes
    B, _, E = edges.shape
    H = hops.shape[0]
    edtype = edges.dtype

    info = plsc.get_sparse_core_info()
    NC, L = info.num_cores, info.num_lanes

    base = T.astype(jnp.int32) + taus.astype(jnp.int32) - _TAU
    scal = jnp.concatenate([base, hops.astype(jnp.int32)])
    params = jnp.broadcast_to(scal[:, None], (B + H, L))

    sc_k = _build_sc_edges_kernel(B, E, H, L, NC)
    edges_out = sc_k(edges.astype(jnp.int32), params)
    weights_out = _build_tc_weights_kernel(B, E, H, weights.dtype)(weights)
    return edges_out.astype(edtype), weights_out
